# baseline (speedup 1.0000x reference)
import functools

import jax
import jax.numpy as jnp
from jax import lax
from jax.experimental import pallas as pl
from jax.experimental.pallas import tpu as pltpu
from jax.experimental.pallas import tpu_sc as plsc

K_FRAC = 0.1

NC, NS, L = 2, 16, 16
NW = NC * NS

ROWS = 16384
D = 2048
ROWS_PER_W = ROWS // NW
GROUPS = ROWS_PER_W // L


def _sc_body(x_hbm, o_hbm, xbuf, hist, dma_sem):
    k = max(1, int(D * K_FRAC))
    wid = lax.axis_index("s") * NC + lax.axis_index("c")
    lane = lax.iota(jnp.int32, L)
    ones = jnp.ones((L,), jnp.int32)
    zeros = jnp.zeros((L,), jnp.int32)
    kv = jnp.full((L,), k, jnp.int32)

    def _z(i, c):
        hist[pl.ds(i * L, L)] = zeros
        return c
    lax.fori_loop(0, 256, _z, 0)

    def monotone_u(v):
        bits = plsc.bitcast(v, jnp.int32)
        return jnp.where(bits < 0, ~bits, bits | jnp.int32(-2**31))

    lane_base = lane * D

    def hist_pass(shift, prefix_shift, prefix):
        def body(j, c):
            v = plsc.load_gather(xbuf, [lane_base + j])
            u = monotone_u(v)
            digit = lax.shift_right_logical(u, shift) & 0xFF
            addr = (digit << 4) | lane
            if prefix is None:
                plsc.addupdate_scatter(hist, [addr], ones)
            else:
                match = lax.shift_right_logical(u, prefix_shift) == prefix
                plsc.addupdate_scatter(hist, [addr], ones, mask=match)
            return c
        lax.fori_loop(0, D, body, 0)

    def select(kcur):
        def body(i, carry):
            acc, chosen, cntgt = carry
            d = 255 - i
            h = hist[pl.ds(d * L, L)]
            hist[pl.ds(d * L, L)] = zeros
            acc2 = acc + h
            newly = (acc < kcur) & (acc2 >= kcur)
            dv = jnp.full((L,), d, jnp.int32)
            chosen = jnp.where(newly, dv, chosen)
            cntgt = jnp.where(newly, acc, cntgt)
            return acc2, chosen, cntgt
        acc, chosen, cntgt = lax.fori_loop(
            0, 256, body, (zeros, zeros, zeros))
        return chosen, kcur - cntgt

    def group(g, c):
        base = (wid * ROWS_PER_W + g * L) * D
        pltpu.sync_copy(x_hbm.at[pl.ds(base, L * D)], xbuf)

        hist_pass(24, 0, None)
        d1, k2 = select(kv)
        hist_pass(16, 24, d1)
        d2, k3 = select(k2)
        pref16 = (d1 << 8) | d2
        hist_pass(8, 16, pref16)
        d3, _ = select(k3)

        t_u = ((pref16 << 8) | d3) << 8
        tbits = jnp.where(t_u < 0, t_u & jnp.int32(0x7FFFFFFF), ~t_u)
        t = plsc.bitcast(tbits, jnp.float32)

        def apply(j, cc):
            idx = lane_base + j
            v = plsc.load_gather(xbuf, [idx])
            y = v / (1.0 + jnp.exp(t - v))
            plsc.store_scatter(xbuf, [idx], y)
            return cc
        lax.fori_loop(0, D, apply, 0)

        pltpu.sync_copy(xbuf, o_hbm.at[pl.ds(base, L * D)])
        return c

    lax.fori_loop(0, GROUPS, group, 0)


def kernel(x):
    b, s, d = x.shape
    xf = x.reshape(b * s * d)
    mesh = plsc.VectorSubcoreMesh(
        core_axis_name="c", subcore_axis_name="s",
        num_cores=NC, num_subcores=NS)
    fn = pl.kernel(
        _sc_body,
        out_type=jax.ShapeDtypeStruct((ROWS * D,), jnp.float32),
        mesh=mesh,
        compiler_params=pltpu.CompilerParams(needs_layout_passes=False),
        scratch_types=[
            pltpu.VMEM((L * D,), jnp.float32),
            pltpu.VMEM((256 * L,), jnp.int32),
            pltpu.SemaphoreType.DMA,
        ],
    )
    return fn(xf).reshape(b, s, d)


# speedup vs baseline: 2.4440x; 2.4440x over previous
import jax
import jax.numpy as jnp
from jax import lax
from jax.experimental import pallas as pl
from jax.experimental.pallas import tpu as pltpu
from jax.experimental.pallas import tpu_sc as plsc

K_FRAC = 0.1

NC, NS, L = 2, 16, 16
NW = NC * NS

ROWS = 16384
D = 2048
ROWS_PER_W = ROWS // NW
GROUPS = ROWS_PER_W // L

IMIN = -2**31


def _sc_body(x_hbm, o_hbm, xbuf, hist, dma_sem):
    k = max(1, int(D * K_FRAC))
    wid = lax.axis_index("s") * NC + lax.axis_index("c")
    lane = lax.iota(jnp.int32, L)
    ones = jnp.ones((L,), jnp.int32)
    zeros = jnp.zeros((L,), jnp.int32)
    kv = jnp.full((L,), k, jnp.int32)
    lane_base = lane * D

    @plsc.parallel_loop(0, 256, unroll=8)
    def _z(i):
        hist[pl.ds(i * L, L)] = zeros

    def select(kcur):
        @plsc.parallel_loop(0, 256, unroll=8, carry=(zeros, zeros, zeros))
        def res(i, carry):
            acc, chosen, cntgt = carry
            d = 255 - i
            h = hist[pl.ds(d * L, L)]
            hist[pl.ds(d * L, L)] = zeros
            acc2 = acc + h
            newly = (acc < kcur) & (acc2 >= kcur)
            dv = jnp.full((L,), d, jnp.int32)
            chosen = jnp.where(newly, dv, chosen)
            cntgt = jnp.where(newly, acc, cntgt)
            return acc2, chosen, cntgt
        acc, chosen, cntgt = res
        return chosen, kcur - cntgt

    def group(g, c):
        base = (wid * ROWS_PER_W + g * L) * D
        pltpu.sync_copy(x_hbm.at[pl.ds(base, L * D)], xbuf)

        @plsc.parallel_loop(0, D, unroll=8)
        def _p1(j):
            idx = lane_base + j
            bits = plsc.bitcast(plsc.load_gather(xbuf, [idx]), jnp.int32)
            u = bits ^ (lax.shift_right_arithmetic(bits, 31) | IMIN)
            plsc.store_scatter(xbuf, [idx], plsc.bitcast(u, jnp.float32))
            addr = (lax.shift_right_logical(u, 20) & 0xFF0) | lane
            plsc.addupdate_scatter(hist, [addr], ones)

        d1, k2 = select(kv)

        @plsc.parallel_loop(0, D, unroll=8)
        def _p2(j):
            u = plsc.bitcast(plsc.load_gather(xbuf, [lane_base + j]),
                             jnp.int32)
            match = lax.shift_right_logical(u, 24) == d1
            addr = (lax.shift_right_logical(u, 12) & 0xFF0) | lane
            plsc.addupdate_scatter(hist, [addr], ones, mask=match)

        d2, k3 = select(k2)
        pref16 = (d1 << 8) | d2

        @plsc.parallel_loop(0, D, unroll=8)
        def _p3(j):
            u = plsc.bitcast(plsc.load_gather(xbuf, [lane_base + j]),
                             jnp.int32)
            match = lax.shift_right_logical(u, 16) == pref16
            addr = (lax.shift_right_logical(u, 4) & 0xFF0) | lane
            plsc.addupdate_scatter(hist, [addr], ones, mask=match)

        d3, _ = select(k3)

        t_u = ((pref16 << 8) | d3) << 8
        tbits = t_u ^ (~lax.shift_right_arithmetic(t_u, 31) | IMIN)
        t = plsc.bitcast(tbits, jnp.float32)

        @plsc.parallel_loop(0, D, unroll=8)
        def _ap(j):
            idx = lane_base + j
            u = plsc.bitcast(plsc.load_gather(xbuf, [idx]), jnp.int32)
            xb = u ^ (~lax.shift_right_arithmetic(u, 31) | IMIN)
            v = plsc.bitcast(xb, jnp.float32)
            y = v / (1.0 + jnp.exp(t - v))
            plsc.store_scatter(xbuf, [idx], y)

        pltpu.sync_copy(xbuf, o_hbm.at[pl.ds(base, L * D)])
        return c

    lax.fori_loop(0, GROUPS, group, 0)


def kernel(x):
    b, s, d = x.shape
    xf = x.reshape(b * s * d)
    mesh = plsc.VectorSubcoreMesh(
        core_axis_name="c", subcore_axis_name="s",
        num_cores=NC, num_subcores=NS)
    fn = pl.kernel(
        _sc_body,
        out_type=jax.ShapeDtypeStruct((ROWS * D,), jnp.float32),
        mesh=mesh,
        compiler_params=pltpu.CompilerParams(needs_layout_passes=False),
        scratch_types=[
            pltpu.VMEM((L * D,), jnp.float32),
            pltpu.VMEM((256 * L,), jnp.int32),
            pltpu.SemaphoreType.DMA,
        ],
    )
    return fn(xf).reshape(b, s, d)


# speedup vs baseline: 7.2761x; 2.9771x over previous
import jax
import jax.numpy as jnp
from jax import lax
from jax.experimental import pallas as pl
from jax.experimental.pallas import tpu as pltpu
from jax.experimental.pallas import tpu_sc as plsc

K_FRAC = 0.1

NC, NS, L = 2, 16, 16
NW = NC * NS

ROWS = 16384
D = 2048
ROWS_PER_W = ROWS // NW
GROUPS = ROWS_PER_W // L

IMIN = -2**31


def _sc_body(x_hbm, o_hbm, xbuf, hist, dma_sem):
    k = max(1, int(D * K_FRAC))
    wid = lax.axis_index("s") * NC + lax.axis_index("c")
    lane = lax.iota(jnp.int32, L)
    ones = jnp.ones((L,), jnp.int32)
    zeros = jnp.zeros((L,), jnp.int32)
    kv = jnp.full((L,), k, jnp.int32)
    lane_base = lane * D

    @plsc.parallel_loop(0, 256, unroll=8)
    def _z(i):
        hist[pl.ds(i * L, L)] = zeros

    def select(kcur):
        @plsc.parallel_loop(0, 256, unroll=8, carry=(zeros, zeros, zeros))
        def res(i, carry):
            acc, chosen, cntgt = carry
            d = 255 - i
            h = hist[pl.ds(d * L, L)]
            hist[pl.ds(d * L, L)] = zeros
            acc2 = acc + h
            newly = (acc < kcur) & (acc2 >= kcur)
            dv = jnp.full((L,), d, jnp.int32)
            chosen = jnp.where(newly, dv, chosen)
            cntgt = jnp.where(newly, acc, cntgt)
            return acc2, chosen, cntgt
        acc, chosen, cntgt = res
        return chosen, kcur - cntgt

    def group(g, c):
        base = (wid * ROWS_PER_W + g * L) * D
        pltpu.sync_copy(x_hbm.at[pl.ds(base, L * D)], xbuf)

        @plsc.parallel_loop(0, D, unroll=8)
        def _p1(j):
            idx = lane_base + ((j + lane) & (D - 1))
            bits = plsc.bitcast(plsc.load_gather(xbuf, [idx]), jnp.int32)
            u = bits ^ (lax.shift_right_arithmetic(bits, 31) | IMIN)
            plsc.store_scatter(xbuf, [idx], plsc.bitcast(u, jnp.float32))
            addr = (lax.shift_right_logical(u, 20) & 0xFF0) | lane
            plsc.addupdate_scatter(hist, [addr], ones)

        d1, k2 = select(kv)

        @plsc.parallel_loop(0, D, unroll=8)
        def _p2(j):
            idx = lane_base + ((j + lane) & (D - 1))
            u = plsc.bitcast(plsc.load_gather(xbuf, [idx]), jnp.int32)
            match = lax.shift_right_logical(u, 24) == d1
            addr = (lax.shift_right_logical(u, 12) & 0xFF0) | lane
            plsc.addupdate_scatter(hist, [addr], ones, mask=match)

        d2, k3 = select(k2)
        pref16 = (d1 << 8) | d2

        @plsc.parallel_loop(0, D, unroll=8)
        def _p3(j):
            idx = lane_base + ((j + lane) & (D - 1))
            u = plsc.bitcast(plsc.load_gather(xbuf, [idx]), jnp.int32)
            match = lax.shift_right_logical(u, 16) == pref16
            addr = (lax.shift_right_logical(u, 4) & 0xFF0) | lane
            plsc.addupdate_scatter(hist, [addr], ones, mask=match)

        d3, _ = select(k3)

        t_u = ((pref16 << 8) | d3) << 8
        tbits = t_u ^ (~lax.shift_right_arithmetic(t_u, 31) | IMIN)
        t = plsc.bitcast(tbits, jnp.float32)

        @plsc.parallel_loop(0, D, unroll=8)
        def _ap(j):
            idx = lane_base + ((j + lane) & (D - 1))
            u = plsc.bitcast(plsc.load_gather(xbuf, [idx]), jnp.int32)
            xb = u ^ (~lax.shift_right_arithmetic(u, 31) | IMIN)
            v = plsc.bitcast(xb, jnp.float32)
            y = v / (1.0 + jnp.exp(t - v))
            plsc.store_scatter(xbuf, [idx], y)

        pltpu.sync_copy(xbuf, o_hbm.at[pl.ds(base, L * D)])
        return c

    lax.fori_loop(0, GROUPS, group, 0)


def kernel(x):
    b, s, d = x.shape
    xf = x.reshape(b * s * d)
    mesh = plsc.VectorSubcoreMesh(
        core_axis_name="c", subcore_axis_name="s",
        num_cores=NC, num_subcores=NS)
    fn = pl.kernel(
        _sc_body,
        out_type=jax.ShapeDtypeStruct((ROWS * D,), jnp.float32),
        mesh=mesh,
        compiler_params=pltpu.CompilerParams(needs_layout_passes=False),
        scratch_types=[
            pltpu.VMEM((L * D,), jnp.float32),
            pltpu.VMEM((256 * L,), jnp.int32),
            pltpu.SemaphoreType.DMA,
        ],
    )
    return fn(xf).reshape(b, s, d)


# speedup vs baseline: 8.2830x; 1.1384x over previous
import jax
import jax.numpy as jnp
from jax import lax
from jax.experimental import pallas as pl
from jax.experimental.pallas import tpu as pltpu
from jax.experimental.pallas import tpu_sc as plsc

K_FRAC = 0.1

NC, NS, L = 2, 16, 16
NW = NC * NS

ROWS = 16384
D = 2048
ROWS_PER_W = ROWS // NW
GROUPS = ROWS_PER_W // L

IMIN = -2**31


def _sc_body(x_hbm, o_hbm, xb0, xb1, hist, isem0, isem1, osem0, osem1):
    k = max(1, int(D * K_FRAC))
    wid = lax.axis_index("s") * NC + lax.axis_index("c")
    lane = lax.iota(jnp.int32, L)
    ones = jnp.ones((L,), jnp.int32)
    zeros = jnp.zeros((L,), jnp.int32)
    kv = jnp.full((L,), k, jnp.int32)
    lane_base = lane * D
    row0 = wid * ROWS_PER_W

    bufs = (xb0, xb1)
    isems = (isem0, isem1)
    osems = (osem0, osem1)

    def start_in(g, b):
        return pltpu.async_copy(
            x_hbm.at[pl.ds((row0 + g * L) * D, L * D)], bufs[b], isems[b])

    def start_out(g, b):
        return pltpu.async_copy(
            bufs[b], o_hbm.at[pl.ds((row0 + g * L) * D, L * D)], osems[b])

    @plsc.parallel_loop(0, 256, unroll=8)
    def _z(i):
        hist[pl.ds(i * L, L)] = zeros

    def select(kcur):
        @plsc.parallel_loop(0, 256, unroll=8, carry=(zeros, zeros, zeros))
        def res(i, carry):
            acc, chosen, cntgt = carry
            d = 255 - i
            h = hist[pl.ds(d * L, L)]
            hist[pl.ds(d * L, L)] = zeros
            acc2 = acc + h
            newly = (acc < kcur) & (acc2 >= kcur)
            dv = jnp.full((L,), d, jnp.int32)
            chosen = jnp.where(newly, dv, chosen)
            cntgt = jnp.where(newly, acc, cntgt)
            return acc2, chosen, cntgt
        acc, chosen, cntgt = res
        return chosen, kcur - cntgt

    def compute(g, b, mid):
        xbuf = bufs[b]

        @plsc.parallel_loop(0, D, unroll=8)
        def _p1(j):
            idx = lane_base + ((j + lane) & (D - 1))
            bits = plsc.bitcast(plsc.load_gather(xbuf, [idx]), jnp.int32)
            u = bits ^ (lax.shift_right_arithmetic(bits, 31) | IMIN)
            plsc.store_scatter(xbuf, [idx], plsc.bitcast(u, jnp.float32))
            addr = (lax.shift_right_logical(u, 20) & 0xFF0) | lane
            plsc.addupdate_scatter(hist, [addr], ones)

        mid()

        d1, k2 = select(kv)

        @plsc.parallel_loop(0, D, unroll=8)
        def _p2(j):
            idx = lane_base + ((j + lane) & (D - 1))
            u = plsc.bitcast(plsc.load_gather(xbuf, [idx]), jnp.int32)
            match = lax.shift_right_logical(u, 24) == d1
            addr = (lax.shift_right_logical(u, 12) & 0xFF0) | lane
            plsc.addupdate_scatter(hist, [addr], ones, mask=match)

        d2, k3 = select(k2)
        pref16 = (d1 << 8) | d2

        @plsc.parallel_loop(0, D, unroll=8)
        def _p3(j):
            idx = lane_base + ((j + lane) & (D - 1))
            u = plsc.bitcast(plsc.load_gather(xbuf, [idx]), jnp.int32)
            match = lax.shift_right_logical(u, 16) == pref16
            addr = (lax.shift_right_logical(u, 4) & 0xFF0) | lane
            plsc.addupdate_scatter(hist, [addr], ones, mask=match)

        d3, _ = select(k3)

        t_u = ((pref16 << 8) | d3) << 8
        tbits = t_u ^ (~lax.shift_right_arithmetic(t_u, 31) | IMIN)
        t = plsc.bitcast(tbits, jnp.float32)

        @plsc.parallel_loop(0, D, unroll=8)
        def _ap(j):
            idx = lane_base + ((j + lane) & (D - 1))
            u = plsc.bitcast(plsc.load_gather(xbuf, [idx]), jnp.int32)
            xb = u ^ (~lax.shift_right_arithmetic(u, 31) | IMIN)
            v = plsc.bitcast(xb, jnp.float32)
            y = v / (1.0 + jnp.exp(t - v))
            plsc.store_scatter(xbuf, [idx], y)

    def wait_in(g, b):
        pltpu.make_async_copy(
            x_hbm.at[pl.ds((row0 + g * L) * D, L * D)], bufs[b],
            isems[b]).wait()

    def wait_out(g, b):
        pltpu.make_async_copy(
            bufs[b], o_hbm.at[pl.ds((row0 + g * L) * D, L * D)],
            osems[b]).wait()

    start_in(0, 0)
    start_in(1, 1)

    def pair(h, c):
        wait_in(2 * h, 0)

        def mid0():
            @pl.when(h > 0)
            def _():
                wait_out(2 * h - 1, 1)
                start_in(2 * h + 1, 1)

        compute(2 * h, 0, mid0)
        start_out(2 * h, 0)

        def mid1():
            @pl.when(h + 1 < GROUPS // 2)
            def _():
                wait_out(2 * h, 0)
                start_in(2 * h + 2, 0)

        wait_in(2 * h + 1, 1)
        compute(2 * h + 1, 1, mid1)
        start_out(2 * h + 1, 1)

        @pl.when(h + 1 >= GROUPS // 2)
        def _():
            wait_out(2 * h, 0)
        return c

    lax.fori_loop(0, GROUPS // 2, pair, 0)
    wait_out(GROUPS - 1, 1)


def kernel(x):
    b, s, d = x.shape
    xf = x.reshape(b * s * d)
    mesh = plsc.VectorSubcoreMesh(
        core_axis_name="c", subcore_axis_name="s",
        num_cores=NC, num_subcores=NS)
    fn = pl.kernel(
        _sc_body,
        out_type=jax.ShapeDtypeStruct((ROWS * D,), jnp.float32),
        mesh=mesh,
        compiler_params=pltpu.CompilerParams(needs_layout_passes=False),
        scratch_types=[
            pltpu.VMEM((L * D,), jnp.float32),
            pltpu.VMEM((L * D,), jnp.float32),
            pltpu.VMEM((256 * L,), jnp.int32),
            pltpu.SemaphoreType.DMA,
            pltpu.SemaphoreType.DMA,
            pltpu.SemaphoreType.DMA,
            pltpu.SemaphoreType.DMA,
        ],
    )
    return fn(xf).reshape(b, s, d)


# speedup vs baseline: 9.5451x; 1.1524x over previous
import jax
import jax.numpy as jnp
from jax import lax
from jax.experimental import pallas as pl
from jax.experimental.pallas import tpu as pltpu
from jax.experimental.pallas import tpu_sc as plsc

K_FRAC = 0.1

NC, NS, L = 2, 16, 16
NW = NC * NS

ROWS = 16384
D = 2048
ROWS_PER_W = ROWS // NW
GROUPS = ROWS_PER_W // L

IMIN = -2**31


def _sc_body(x_hbm, o_hbm, xb0, xb1, hist, isem0, isem1, osem0, osem1):
    k = max(1, int(D * K_FRAC))
    wid = lax.axis_index("s") * NC + lax.axis_index("c")
    lane = lax.iota(jnp.int32, L)
    ones = jnp.ones((L,), jnp.int32)
    zeros = jnp.zeros((L,), jnp.int32)
    kv = jnp.full((L,), k, jnp.int32)
    lane_base = lane * D
    row0 = wid * ROWS_PER_W

    bufs = (xb0, xb1)
    isems = (isem0, isem1)
    osems = (osem0, osem1)

    def start_in(g, b):
        return pltpu.async_copy(
            x_hbm.at[pl.ds((row0 + g * L), L), :], bufs[b], isems[b])

    def start_out(g, b):
        return pltpu.async_copy(
            bufs[b], o_hbm.at[pl.ds((row0 + g * L), L), :], osems[b])

    @plsc.parallel_loop(0, 256, unroll=8)
    def _z(i):
        hist[pl.ds(i * L, L)] = zeros

    def select(kcur):
        @plsc.parallel_loop(0, 256, unroll=8, carry=(zeros, zeros, zeros))
        def res(i, carry):
            acc, chosen, cntgt = carry
            d = 255 - i
            h = hist[pl.ds(d * L, L)]
            hist[pl.ds(d * L, L)] = zeros
            acc2 = acc + h
            newly = (acc < kcur) & (acc2 >= kcur)
            dv = jnp.full((L,), d, jnp.int32)
            chosen = jnp.where(newly, dv, chosen)
            cntgt = jnp.where(newly, acc, cntgt)
            return acc2, chosen, cntgt
        acc, chosen, cntgt = res
        return chosen, kcur - cntgt

    def compute(g, b, mid):
        xbuf = bufs[b]

        @plsc.parallel_loop(0, D, unroll=8)
        def _p1(j):
            col = (j + lane) & (D - 1)
            bits = plsc.bitcast(
                plsc.load_gather(xbuf, [lane, col]), jnp.int32)
            u = bits ^ (lax.shift_right_arithmetic(bits, 31) | IMIN)
            plsc.store_scatter(xbuf, [lane, col],
                               plsc.bitcast(u, jnp.float32))
            addr = (lax.shift_right_logical(u, 20) & 0xFF0) | lane
            plsc.addupdate_scatter(hist, [addr], ones)

        mid()

        d1, k2 = select(kv)

        @plsc.parallel_loop(0, D, unroll=8)
        def _p2(j):
            col = (j + lane) & (D - 1)
            u = plsc.bitcast(plsc.load_gather(xbuf, [lane, col]), jnp.int32)
            match = lax.shift_right_logical(u, 24) == d1
            addr = (lax.shift_right_logical(u, 12) & 0xFF0) | lane
            plsc.addupdate_scatter(hist, [addr], ones, mask=match)

        d2, k3 = select(k2)
        pref16 = (d1 << 8) | d2

        @plsc.parallel_loop(0, D, unroll=8)
        def _p3(j):
            col = (j + lane) & (D - 1)
            u = plsc.bitcast(plsc.load_gather(xbuf, [lane, col]), jnp.int32)
            match = lax.shift_right_logical(u, 16) == pref16
            addr = (lax.shift_right_logical(u, 4) & 0xFF0) | lane
            plsc.addupdate_scatter(hist, [addr], ones, mask=match)

        d3, _ = select(k3)

        t_u = ((pref16 << 8) | d3) << 8
        tbits = t_u ^ (~lax.shift_right_arithmetic(t_u, 31) | IMIN)
        t = plsc.bitcast(tbits, jnp.float32)

        @plsc.parallel_loop(0, D, unroll=8)
        def _ap(j):
            col = (j + lane) & (D - 1)
            u = plsc.bitcast(plsc.load_gather(xbuf, [lane, col]), jnp.int32)
            xb = u ^ (~lax.shift_right_arithmetic(u, 31) | IMIN)
            v = plsc.bitcast(xb, jnp.float32)
            y = v / (1.0 + jnp.exp(t - v))
            plsc.store_scatter(xbuf, [lane, col], y)

    def wait_in(g, b):
        pltpu.make_async_copy(
            x_hbm.at[pl.ds((row0 + g * L), L), :], bufs[b],
            isems[b]).wait()

    def wait_out(g, b):
        pltpu.make_async_copy(
            bufs[b], o_hbm.at[pl.ds((row0 + g * L), L), :],
            osems[b]).wait()

    start_in(0, 0)
    start_in(1, 1)

    def pair(h, c):
        wait_in(2 * h, 0)

        def mid0():
            @pl.when(h > 0)
            def _():
                wait_out(2 * h - 1, 1)
                start_in(2 * h + 1, 1)

        compute(2 * h, 0, mid0)
        start_out(2 * h, 0)

        def mid1():
            @pl.when(h + 1 < GROUPS // 2)
            def _():
                wait_out(2 * h, 0)
                start_in(2 * h + 2, 0)

        wait_in(2 * h + 1, 1)
        compute(2 * h + 1, 1, mid1)
        start_out(2 * h + 1, 1)

        @pl.when(h + 1 >= GROUPS // 2)
        def _():
            wait_out(2 * h, 0)
        return c

    lax.fori_loop(0, GROUPS // 2, pair, 0)
    wait_out(GROUPS - 1, 1)


def kernel(x):
    b, s, d = x.shape
    xf = x.reshape(b * s, d)
    mesh = plsc.VectorSubcoreMesh(
        core_axis_name="c", subcore_axis_name="s",
        num_cores=NC, num_subcores=NS)
    fn = pl.kernel(
        _sc_body,
        out_type=jax.ShapeDtypeStruct((ROWS, D), jnp.float32),
        mesh=mesh,
        compiler_params=pltpu.CompilerParams(needs_layout_passes=False),
        scratch_types=[
            pltpu.VMEM((L, D), jnp.float32),
            pltpu.VMEM((L, D), jnp.float32),
            pltpu.VMEM((256 * L,), jnp.int32),
            pltpu.SemaphoreType.DMA,
            pltpu.SemaphoreType.DMA,
            pltpu.SemaphoreType.DMA,
            pltpu.SemaphoreType.DMA,
        ],
    )
    return fn(xf).reshape(b, s, d)


# speedup vs baseline: 11.8523x; 1.2417x over previous
import jax
import jax.numpy as jnp
from jax import lax
from jax.experimental import pallas as pl
from jax.experimental.pallas import tpu as pltpu
from jax.experimental.pallas import tpu_sc as plsc

K_FRAC = 0.1

NC, NS, L = 2, 16, 16
NW = NC * NS

ROWS = 16384
D = 2048
ROWS_PER_W = ROWS // NW
GROUPS = ROWS_PER_W // L

IMIN = -2**31


def _sc_body(x_hbm, o_hbm, xb0, xb1, hist, isem0, isem1, osem0, osem1):
    k = max(1, int(D * K_FRAC))
    wid = lax.axis_index("s") * NC + lax.axis_index("c")
    lane = lax.iota(jnp.int32, L)
    ones = jnp.ones((L,), jnp.int32)
    zeros = jnp.zeros((L,), jnp.int32)
    kv = jnp.full((L,), k, jnp.int32)
    lane_base = lane * D
    row0 = wid * ROWS_PER_W

    bufs = (xb0, xb1)
    isems = (isem0, isem1)
    osems = (osem0, osem1)

    def start_in(g, b):
        return pltpu.async_copy(
            x_hbm.at[pl.ds((row0 + g * L), L), :], bufs[b], isems[b])

    def start_out(g, b):
        return pltpu.async_copy(
            bufs[b], o_hbm.at[pl.ds((row0 + g * L), L), :], osems[b])

    @plsc.parallel_loop(0, 512, unroll=8)
    def _z(i):
        hist[pl.ds(i * L, L)] = zeros

    def select(kcur):
        @plsc.parallel_loop(0, 512, unroll=8, carry=(zeros, zeros, zeros))
        def res(i, carry):
            acc, chosen, cntgt = carry
            d = 511 - i
            h = hist[pl.ds(d * L, L)]
            hist[pl.ds(d * L, L)] = zeros
            acc2 = acc + h
            newly = (acc < kcur) & (acc2 >= kcur)
            dv = jnp.full((L,), d, jnp.int32)
            chosen = jnp.where(newly, dv, chosen)
            cntgt = jnp.where(newly, acc, cntgt)
            return acc2, chosen, cntgt
        acc, chosen, cntgt = res
        return chosen, kcur - cntgt

    def compute(g, b, mid):
        xbuf = bufs[b]

        @plsc.parallel_loop(0, D, unroll=8)
        def _p1(j):
            col = (j + lane) & (D - 1)
            bits = plsc.bitcast(
                plsc.load_gather(xbuf, [lane, col]), jnp.int32)
            u = bits ^ (lax.shift_right_arithmetic(bits, 31) | IMIN)
            plsc.store_scatter(xbuf, [lane, col],
                               plsc.bitcast(u, jnp.float32))
            addr = (lax.shift_right_logical(u, 19) & 0x1FF0) | lane
            plsc.addupdate_scatter(hist, [addr], ones)

        mid()

        d1, k2 = select(kv)

        @plsc.parallel_loop(0, D, unroll=8)
        def _p2(j):
            col = (j + lane) & (D - 1)
            u = plsc.bitcast(plsc.load_gather(xbuf, [lane, col]), jnp.int32)
            match = lax.shift_right_logical(u, 23) == d1
            addr = (lax.shift_right_logical(u, 10) & 0x1FF0) | lane
            plsc.addupdate_scatter(hist, [addr], ones, mask=match)

        d2, _ = select(k2)

        t_u = ((d1 << 9) | d2) << 14
        tbits = t_u ^ (~lax.shift_right_arithmetic(t_u, 31) | IMIN)
        t = plsc.bitcast(tbits, jnp.float32)

        @plsc.parallel_loop(0, D, unroll=8)
        def _ap(j):
            col = (j + lane) & (D - 1)
            u = plsc.bitcast(plsc.load_gather(xbuf, [lane, col]), jnp.int32)
            xb = u ^ (~lax.shift_right_arithmetic(u, 31) | IMIN)
            v = plsc.bitcast(xb, jnp.float32)
            y = v / (1.0 + jnp.exp(t - v))
            plsc.store_scatter(xbuf, [lane, col], y)

    def wait_in(g, b):
        pltpu.make_async_copy(
            x_hbm.at[pl.ds((row0 + g * L), L), :], bufs[b],
            isems[b]).wait()

    def wait_out(g, b):
        pltpu.make_async_copy(
            bufs[b], o_hbm.at[pl.ds((row0 + g * L), L), :],
            osems[b]).wait()

    start_in(0, 0)
    start_in(1, 1)

    def pair(h, c):
        wait_in(2 * h, 0)

        def mid0():
            @pl.when(h > 0)
            def _():
                wait_out(2 * h - 1, 1)
                start_in(2 * h + 1, 1)

        compute(2 * h, 0, mid0)
        start_out(2 * h, 0)

        def mid1():
            @pl.when(h + 1 < GROUPS // 2)
            def _():
                wait_out(2 * h, 0)
                start_in(2 * h + 2, 0)

        wait_in(2 * h + 1, 1)
        compute(2 * h + 1, 1, mid1)
        start_out(2 * h + 1, 1)

        @pl.when(h + 1 >= GROUPS // 2)
        def _():
            wait_out(2 * h, 0)
        return c

    lax.fori_loop(0, GROUPS // 2, pair, 0)
    wait_out(GROUPS - 1, 1)


def kernel(x):
    b, s, d = x.shape
    xf = x.reshape(b * s, d)
    mesh = plsc.VectorSubcoreMesh(
        core_axis_name="c", subcore_axis_name="s",
        num_cores=NC, num_subcores=NS)
    fn = pl.kernel(
        _sc_body,
        out_type=jax.ShapeDtypeStruct((ROWS, D), jnp.float32),
        mesh=mesh,
        compiler_params=pltpu.CompilerParams(needs_layout_passes=False),
        scratch_types=[
            pltpu.VMEM((L, D), jnp.float32),
            pltpu.VMEM((L, D), jnp.float32),
            pltpu.VMEM((512 * L,), jnp.int32),
            pltpu.SemaphoreType.DMA,
            pltpu.SemaphoreType.DMA,
            pltpu.SemaphoreType.DMA,
            pltpu.SemaphoreType.DMA,
        ],
    )
    return fn(xf).reshape(b, s, d)
